# baseline (speedup 1.0000x reference)
import functools
import math

import jax
import jax.numpy as jnp
from jax.experimental import pallas as pl
from jax.experimental.pallas import tpu as pltpu
from jax.experimental.pallas import tpu_sc as plsc

_N = 10000
_NP = 10240
_R = 128
_CB = 512
_NCH = _NP // _CB
_K = 20


def _knn_body(c0_ref, c1_ref, xr_ref, br_ref, xc_ref, sqc_ref, bc_ref,
              idx_ref, d_ref, m_ref):
    g = pl.program_id(0)
    c0 = c0_ref[g]
    c1 = c1_ref[g]
    xr = xr_ref[...]
    sqr = jnp.sum(xr * xr, axis=1, keepdims=True)
    br = br_ref[...]
    m_ref[...] = jnp.full((_NCH, _R, 1), jnp.inf, jnp.float32)

    def fill(ci, carry):
        xc = xc_ref[ci]
        dd = sqr + sqc_ref[ci] - 2.0 * jnp.dot(
            xr, xc, preferred_element_type=jnp.float32)
        dd = jnp.where(br == bc_ref[ci], dd, jnp.inf)
        d_ref[ci] = dd
        m_ref[ci] = jnp.min(dd, axis=1, keepdims=True)
        return carry

    jax.lax.fori_loop(c0, c1, fill, 0)

    iotaf = jax.lax.broadcasted_iota(
        jnp.int32, (1, _CB), 1).astype(jnp.float32)
    bigf = jnp.float32(2.0 * _NP)
    outs = []
    for _ in range(_K):
        cm = jnp.min(m_ref[...], axis=0)

        def extract(ci, bi):
            d = d_ref[ci]
            match = d == cm
            iif = iotaf + ci.astype(jnp.float32) * _CB
            im_c = jnp.min(jnp.where(match, iif, bigf), axis=1, keepdims=True)
            is_first = bi >= bigf
            dnew = jnp.where((iif == im_c) & is_first, jnp.inf, d)
            d_ref[ci] = dnew
            m_ref[ci] = jnp.min(dnew, axis=1, keepdims=True)
            return jnp.minimum(bi, im_c)

        bi = jax.lax.fori_loop(c0, c1, extract, jnp.full((_R, 1), bigf))
        outs.append(bi)
    idx = jnp.concatenate(outs, axis=1).astype(jnp.int32)
    idx_ref[...] = jnp.minimum(idx, _N - 1)


def _knn_idx_pallas(x, batch):
    n, c = x.shape
    cp = max(8, ((c + 7) // 8) * 8)
    xp = jnp.zeros((_NP, cp), jnp.float32).at[:n, :c].set(x)
    batp = jnp.concatenate(
        [batch.astype(jnp.int32), jnp.full((_NP - n,), 8, jnp.int32)])
    batf = batp.astype(jnp.float32)

    starts9 = jnp.searchsorted(batp, jnp.arange(9, dtype=jnp.int32), side="left")
    ends9 = jnp.searchsorted(batp, jnp.arange(9, dtype=jnp.int32), side="right")
    b_lo = batp[0::_R]
    b_hi = batp[_R - 1::_R]
    cs = starts9[b_lo]
    ce = ends9[b_hi]
    c0 = (cs // _CB).astype(jnp.int32)
    c1 = ((ce + _CB - 1) // _CB).astype(jnp.int32)

    x3t = xp.reshape(_NCH, _CB, cp).transpose(0, 2, 1)
    sqc3 = jnp.sum(xp * xp, axis=1).reshape(_NCH, _CB)[:, None, :]
    bc3 = batf.reshape(_NCH, _CB)[:, None, :]
    br2 = batf[:, None]

    grid = _NP // _R
    idx = pl.pallas_call(
        _knn_body,
        grid=(grid,),
        in_specs=[
            pl.BlockSpec(memory_space=pltpu.SMEM),
            pl.BlockSpec(memory_space=pltpu.SMEM),
            pl.BlockSpec((_R, cp), lambda g: (g, 0)),
            pl.BlockSpec((_R, 1), lambda g: (g, 0)),
            pl.BlockSpec((_NCH, cp, _CB), lambda g: (0, 0, 0)),
            pl.BlockSpec((_NCH, 1, _CB), lambda g: (0, 0, 0)),
            pl.BlockSpec((_NCH, 1, _CB), lambda g: (0, 0, 0)),
        ],
        out_specs=pl.BlockSpec((_R, _K), lambda g: (g, 0)),
        out_shape=jax.ShapeDtypeStruct((_NP, _K), jnp.int32),
        scratch_shapes=[pltpu.VMEM((_NCH, _R, _CB), jnp.float32),
                        pltpu.VMEM((_NCH, _R, 1), jnp.float32)],
    )(c0, c1, xp, br2, x3t, sqc3, bc3)
    return idx[:n]


def _sc_gather(table, idx3, h):
    n_st = _NP // 256
    n_super = _K * n_st
    per_w = n_super // 32
    mesh = plsc.VectorSubcoreMesh(core_axis_name="c", subcore_axis_name="s")

    @functools.partial(
        pl.kernel, mesh=mesh,
        out_type=jax.ShapeDtypeStruct((_K, _NP, h), jnp.float32),
        scratch_types=[
            pltpu.VMEM((2, 128), jnp.int32),
            pltpu.VMEM((256, h), jnp.float32),
            pltpu.SemaphoreType.DMA,
        ],
    )
    def gk(table_h, idx_h, out_h, idx_v, rows_v, sem):
        wid = jax.lax.axis_index("s") * 2 + jax.lax.axis_index("c")

        def body(j, carry):
            st = wid * per_w + j
            k = st // n_st
            pos = (st % n_st) * 256
            pltpu.sync_copy(idx_h.at[st], idx_v)
            cps = [
                pltpu.async_copy(
                    table_h.at[idx_v.at[i]],
                    rows_v.at[pl.ds(i * 128, 128)], sem)
                for i in range(2)
            ]
            for cp in cps:
                cp.wait()
            pltpu.sync_copy(rows_v, out_h.at[k, pl.ds(pos, 256)])
            return carry

        jax.lax.fori_loop(0, per_w, body, 0)

    return gk(table, idx3)


def _e_stats1_body(cp, h, g_ref, x_ref, w1_ref, b1_ref, o_ref):
    g = pl.program_id(0)
    xi = x_ref[...]
    w1 = w1_ref[...]
    b1 = b1_ref[...]
    rows = jax.lax.broadcasted_iota(jnp.int32, (_R, 1), 0) + g * _R
    vm = rows < _N
    s = jnp.zeros((1, h), jnp.float32)
    q = jnp.zeros((1, h), jnp.float32)
    for k in range(_K):
        e = jnp.concatenate([xi, g_ref[k] - xi], axis=1)
        y = jnp.dot(e, w1, preferred_element_type=jnp.float32) + b1
        y = jnp.where(vm, y, 0.0)
        s = s + jnp.sum(y, axis=0, keepdims=True)
        q = q + jnp.sum(y * y, axis=0, keepdims=True)
    o_ref[0, 0:1, :] = s
    o_ref[0, 1:2, :] = q


def _e_stats2_body(cp, h, o, g_ref, x_ref, w1_ref, b1_ref, a1_ref, c1_ref,
                   w2_ref, o_ref):
    g = pl.program_id(0)
    xi = x_ref[...]
    w1 = w1_ref[...]
    b1 = b1_ref[...]
    a1 = a1_ref[...]
    c1 = c1_ref[...]
    w2 = w2_ref[...]
    rows = jax.lax.broadcasted_iota(jnp.int32, (_R, 1), 0) + g * _R
    vm = rows < _N
    s = jnp.zeros((1, o), jnp.float32)
    q = jnp.zeros((1, o), jnp.float32)
    for k in range(_K):
        e = jnp.concatenate([xi, g_ref[k] - xi], axis=1)
        y1 = jnp.dot(e, w1, preferred_element_type=jnp.float32) + b1
        z1 = _leaky(y1 * a1 + c1)
        y2 = jnp.where(vm, jnp.dot(z1, w2, preferred_element_type=jnp.float32),
                       0.0)
        s = s + jnp.sum(y2, axis=0, keepdims=True)
        q = q + jnp.sum(y2 * y2, axis=0, keepdims=True)
    o_ref[0, 0:1, :] = s
    o_ref[0, 1:2, :] = q


def _e_final_body(cp, h, o, g_ref, x_ref, w1_ref, b1_ref, a1_ref, c1_ref,
                  w2_ref, a2_ref, c2_ref, out_ref):
    xi = x_ref[...]
    w1 = w1_ref[...]
    b1 = b1_ref[...]
    a1 = a1_ref[...]
    c1 = c1_ref[...]
    w2 = w2_ref[...]
    a2 = a2_ref[...]
    c2 = c2_ref[...]
    acc = jnp.full((_R, o), -jnp.inf, jnp.float32)
    for k in range(_K):
        e = jnp.concatenate([xi, g_ref[k] - xi], axis=1)
        y1 = jnp.dot(e, w1, preferred_element_type=jnp.float32) + b1
        z1 = _leaky(y1 * a1 + c1)
        y2 = jnp.dot(z1, w2, preferred_element_type=jnp.float32)
        acc = jnp.maximum(acc, _leaky(y2 * a2 + c2))
    out_ref[...] = acc


def _edge_conv_sc(x, idx, lay1, lay2):
    n, c = x.shape
    w1, b1, g1, be1 = lay1
    w2, b2, g2, be2 = lay2
    h = w1.shape[1]
    o = w2.shape[1]
    cp = ((c + 127) // 128) * 128
    xp = jnp.pad(x, ((0, _NP - n), (0, cp - c)))
    w1cat = jnp.zeros((2 * cp, h), jnp.float32)
    w1cat = w1cat.at[:c].set(w1[:c]).at[cp:cp + c].set(w1[c:])
    b1r = b1[None, :]
    idxp = jnp.pad(idx, ((0, _NP - n), (0, 0)))
    idx3 = idxp.T.reshape(_K * (_NP // 256), 2, 128)

    g3 = _sc_gather(xp, idx3, cp)

    grid = _NP // _R
    full3 = pl.BlockSpec((_K, _R, cp), lambda g: (0, g, 0))
    rowb = pl.BlockSpec((_R, cp), lambda g: (g, 0))
    vec = lambda d: pl.BlockSpec((1, d), lambda g: (0, 0))
    mat = lambda a, b: pl.BlockSpec((a, b), lambda g: (0, 0))

    s1, q1 = pl.pallas_call(
        functools.partial(_e_stats1_body, cp, h),
        grid=(grid,),
        in_specs=[full3, rowb, mat(2 * cp, h), vec(h)],
        out_specs=pl.BlockSpec((1, 2, h), lambda g: (g, 0, 0)),
        out_shape=jax.ShapeDtypeStruct((grid, 2, h), jnp.float32),
    )(g3, xp, w1cat, b1r).sum(axis=0)

    cnt = jnp.float32(n * _K)
    m1 = s1 / cnt
    v1 = q1 / cnt - m1 * m1
    sc1 = g1 / jnp.sqrt(v1 + 1e-5)
    a1 = sc1[None, :]
    c1f = (be1 - m1 * sc1)[None, :]

    s2, q2 = pl.pallas_call(
        functools.partial(_e_stats2_body, cp, h, o),
        grid=(grid,),
        in_specs=[full3, rowb, mat(2 * cp, h), vec(h), vec(h), vec(h),
                  mat(h, o)],
        out_specs=pl.BlockSpec((1, 2, o), lambda g: (g, 0, 0)),
        out_shape=jax.ShapeDtypeStruct((grid, 2, o), jnp.float32),
    )(g3, xp, w1cat, b1r, a1, c1f, w2).sum(axis=0)

    m2nb = s2 / cnt
    v2 = q2 / cnt - m2nb * m2nb
    sc2 = g2 / jnp.sqrt(v2 + 1e-5)
    a2 = sc2[None, :]
    c2f = ((b2 - (m2nb + b2)) * sc2 + be2)[None, :]

    out = pl.pallas_call(
        functools.partial(_e_final_body, cp, h, o),
        grid=(grid,),
        in_specs=[full3, rowb, mat(2 * cp, h), vec(h), vec(h), vec(h),
                  mat(h, o), vec(o), vec(o)],
        out_specs=pl.BlockSpec((_R, o), lambda g: (g, 0)),
        out_shape=jax.ShapeDtypeStruct((_NP, o), jnp.float32),
    )(g3, xp, w1cat, b1r, a1, c1f, w2, a2, c2f)
    return out[:n]


def _leaky(x):
    return jnp.where(x >= 0, x, 0.2 * x)


def _bn(x, g, b):
    m = jnp.mean(x, axis=0)
    v = jnp.var(x, axis=0)
    return (x - m) / jnp.sqrt(v + 1e-5) * g + b


def _mlp(x, layers, use_bn):
    for lay in layers:
        if use_bn:
            w, b, g, be = lay
            x = _leaky(_bn(x @ w + b, g, be))
        else:
            w, b = lay
            x = _leaky(x @ w + b)
    return x


def _gather_neighbors(x, idx):
    n, c = x.shape
    cp = ((c + 127) // 128) * 128
    xp = jnp.pad(x, ((0, _NP - n), (0, cp - c)))
    idxp = jnp.pad(idx, ((0, _NP - n), (0, 0)))
    idx3 = idxp.T.reshape(_K * (_NP // 256), 2, 128)
    g3 = _sc_gather(xp, idx3, cp)
    return g3[:, :n, :c].transpose(1, 0, 2)


def _edge_conv(x, batch, layers):
    idx = _knn_idx_pallas(x, batch)
    x_j = _gather_neighbors(x, idx)
    x_i = jnp.broadcast_to(x[:, None, :], x_j.shape)
    hh = jnp.concatenate([x_i, x_j - x_i], axis=-1)
    n, k, c = hh.shape
    hh = _mlp(hh.reshape(n * k, c), layers, True)
    return jnp.max(hh.reshape(n, k, -1), axis=1)


def kernel(positions, features, batch_indices, params):
    n_layers, n_batch = 2, 8
    x = _edge_conv(positions, batch_indices, params["t1"])
    x = _mlp(x, params["t2"], True)
    x = jax.ops.segment_max(x, batch_indices, num_segments=n_batch)
    x = _mlp(x, params["t3"], False)
    w4, b4 = params["t4"]
    x = x @ w4 + b4
    x = x[batch_indices].reshape(-1, 3, 3)
    x0 = jnp.einsum('ni,nij->nj', positions, x)
    x = jnp.concatenate([x0, features], axis=-1)
    for i in range(n_layers):
        x_i = _edge_conv(x, batch_indices, params["convs"][i])
        (w1, b1), (w2, b2) = params["lins"][i]
        x_i = jnp.maximum(x_i @ w1 + b1, 0.0) @ w2 + b2
        wt, bt = params["ltrans"][i]
        x = (x @ wt + bt) + x_i
    return x


# speedup vs baseline: 1.2045x; 1.2045x over previous
import functools
import math

import jax
import jax.numpy as jnp
from jax.experimental import pallas as pl
from jax.experimental.pallas import tpu as pltpu
from jax.experimental.pallas import tpu_sc as plsc

_N = 10000
_NP = 10240
_R = 128
_CB = 1024
_NCH = _NP // _CB
_K = 20


def _knn_body(c0_ref, c1_ref, xr_ref, br_ref, xc_ref, sqc_ref, bc_ref,
              idx_ref, d_ref, m_ref):
    g = pl.program_id(0)
    c0 = c0_ref[g]
    c1 = c1_ref[g]
    xr = xr_ref[...]
    sqr = jnp.sum(xr * xr, axis=1, keepdims=True)
    br = br_ref[...]
    m_ref[...] = jnp.full((_NCH, _R, 1), jnp.inf, jnp.float32)

    def fill(ci, carry):
        xc = xc_ref[ci]
        dd = sqr + sqc_ref[ci] - 2.0 * jnp.dot(
            xr, xc, preferred_element_type=jnp.float32)
        dd = jnp.where(br == bc_ref[ci], dd, jnp.inf)
        d_ref[ci] = dd
        m_ref[ci] = jnp.min(dd, axis=1, keepdims=True)
        return carry

    jax.lax.fori_loop(c0, c1, fill, 0)

    iotaf = jax.lax.broadcasted_iota(
        jnp.int32, (1, _CB), 1).astype(jnp.float32)
    bigf = jnp.float32(2.0 * _NP)
    outs = []
    for _ in range(_K):
        cm = jnp.min(m_ref[...], axis=0)

        def extract(ci, bi):
            d = d_ref[ci]
            match = d == cm
            iif = iotaf + ci.astype(jnp.float32) * _CB
            im_c = jnp.min(jnp.where(match, iif, bigf), axis=1, keepdims=True)
            is_first = bi >= bigf
            dnew = jnp.where((iif == im_c) & is_first, jnp.inf, d)
            d_ref[ci] = dnew
            m_ref[ci] = jnp.min(dnew, axis=1, keepdims=True)
            return jnp.minimum(bi, im_c)

        bi = jax.lax.fori_loop(c0, c1, extract, jnp.full((_R, 1), bigf))
        outs.append(bi)
    idx = jnp.concatenate(outs, axis=1).astype(jnp.int32)
    idx_ref[...] = jnp.minimum(idx, _N - 1)


def _knn_idx_pallas(x, batch):
    n, c = x.shape
    cp = max(8, ((c + 7) // 8) * 8)
    xp = jnp.zeros((_NP, cp), jnp.float32).at[:n, :c].set(x)
    batp = jnp.concatenate(
        [batch.astype(jnp.int32), jnp.full((_NP - n,), 8, jnp.int32)])
    batf = batp.astype(jnp.float32)

    starts9 = jnp.searchsorted(batp, jnp.arange(9, dtype=jnp.int32), side="left")
    ends9 = jnp.searchsorted(batp, jnp.arange(9, dtype=jnp.int32), side="right")
    b_lo = batp[0::_R]
    b_hi = batp[_R - 1::_R]
    cs = starts9[b_lo]
    ce = ends9[b_hi]
    c0 = (cs // _CB).astype(jnp.int32)
    c1 = ((ce + _CB - 1) // _CB).astype(jnp.int32)

    x3t = xp.reshape(_NCH, _CB, cp).transpose(0, 2, 1)
    sqc3 = jnp.sum(xp * xp, axis=1).reshape(_NCH, _CB)[:, None, :]
    bc3 = batf.reshape(_NCH, _CB)[:, None, :]
    br2 = batf[:, None]

    grid = _NP // _R
    idx = pl.pallas_call(
        _knn_body,
        grid=(grid,),
        in_specs=[
            pl.BlockSpec(memory_space=pltpu.SMEM),
            pl.BlockSpec(memory_space=pltpu.SMEM),
            pl.BlockSpec((_R, cp), lambda g: (g, 0)),
            pl.BlockSpec((_R, 1), lambda g: (g, 0)),
            pl.BlockSpec((_NCH, cp, _CB), lambda g: (0, 0, 0)),
            pl.BlockSpec((_NCH, 1, _CB), lambda g: (0, 0, 0)),
            pl.BlockSpec((_NCH, 1, _CB), lambda g: (0, 0, 0)),
        ],
        out_specs=pl.BlockSpec((_R, _K), lambda g: (g, 0)),
        out_shape=jax.ShapeDtypeStruct((_NP, _K), jnp.int32),
        scratch_shapes=[pltpu.VMEM((_NCH, _R, _CB), jnp.float32),
                        pltpu.VMEM((_NCH, _R, 1), jnp.float32)],
    )(c0, c1, xp, br2, x3t, sqc3, bc3)
    return idx[:n]


def _sc_gather(table, idx3, h):
    n_st = _NP // 256
    n_super = _K * n_st
    per_w = n_super // 32
    mesh = plsc.VectorSubcoreMesh(core_axis_name="c", subcore_axis_name="s")

    @functools.partial(
        pl.kernel, mesh=mesh,
        out_type=jax.ShapeDtypeStruct((_K, _NP, h), jnp.float32),
        scratch_types=[
            pltpu.VMEM((2, 128), jnp.int32),
            pltpu.VMEM((256, h), jnp.float32),
            pltpu.SemaphoreType.DMA,
        ],
    )
    def gk(table_h, idx_h, out_h, idx_v, rows_v, sem):
        wid = jax.lax.axis_index("s") * 2 + jax.lax.axis_index("c")

        def body(j, carry):
            st = wid * per_w + j
            k = st // n_st
            pos = (st % n_st) * 256
            pltpu.sync_copy(idx_h.at[st], idx_v)
            cps = [
                pltpu.async_copy(
                    table_h.at[idx_v.at[i]],
                    rows_v.at[pl.ds(i * 128, 128)], sem)
                for i in range(2)
            ]
            for cp in cps:
                cp.wait()
            pltpu.sync_copy(rows_v, out_h.at[k, pl.ds(pos, 256)])
            return carry

        jax.lax.fori_loop(0, per_w, body, 0)

    return gk(table, idx3)


def _e_stats1_body(cp, h, g_ref, x_ref, w1_ref, b1_ref, o_ref):
    g = pl.program_id(0)
    xi = x_ref[...]
    w1 = w1_ref[...]
    b1 = b1_ref[...]
    rows = jax.lax.broadcasted_iota(jnp.int32, (_R, 1), 0) + g * _R
    vm = rows < _N
    s = jnp.zeros((1, h), jnp.float32)
    q = jnp.zeros((1, h), jnp.float32)
    for k in range(_K):
        e = jnp.concatenate([xi, g_ref[k] - xi], axis=1)
        y = jnp.dot(e, w1, preferred_element_type=jnp.float32) + b1
        y = jnp.where(vm, y, 0.0)
        s = s + jnp.sum(y, axis=0, keepdims=True)
        q = q + jnp.sum(y * y, axis=0, keepdims=True)
    o_ref[0, 0:1, :] = s
    o_ref[0, 1:2, :] = q


def _e_stats2_body(cp, h, o, g_ref, x_ref, w1_ref, b1_ref, a1_ref, c1_ref,
                   w2_ref, o_ref):
    g = pl.program_id(0)
    xi = x_ref[...]
    w1 = w1_ref[...]
    b1 = b1_ref[...]
    a1 = a1_ref[...]
    c1 = c1_ref[...]
    w2 = w2_ref[...]
    rows = jax.lax.broadcasted_iota(jnp.int32, (_R, 1), 0) + g * _R
    vm = rows < _N
    s = jnp.zeros((1, o), jnp.float32)
    q = jnp.zeros((1, o), jnp.float32)
    for k in range(_K):
        e = jnp.concatenate([xi, g_ref[k] - xi], axis=1)
        y1 = jnp.dot(e, w1, preferred_element_type=jnp.float32) + b1
        z1 = _leaky(y1 * a1 + c1)
        y2 = jnp.where(vm, jnp.dot(z1, w2, preferred_element_type=jnp.float32),
                       0.0)
        s = s + jnp.sum(y2, axis=0, keepdims=True)
        q = q + jnp.sum(y2 * y2, axis=0, keepdims=True)
    o_ref[0, 0:1, :] = s
    o_ref[0, 1:2, :] = q


def _e_final_body(cp, h, o, g_ref, x_ref, w1_ref, b1_ref, a1_ref, c1_ref,
                  w2_ref, a2_ref, c2_ref, out_ref):
    xi = x_ref[...]
    w1 = w1_ref[...]
    b1 = b1_ref[...]
    a1 = a1_ref[...]
    c1 = c1_ref[...]
    w2 = w2_ref[...]
    a2 = a2_ref[...]
    c2 = c2_ref[...]
    acc = jnp.full((_R, o), -jnp.inf, jnp.float32)
    for k in range(_K):
        e = jnp.concatenate([xi, g_ref[k] - xi], axis=1)
        y1 = jnp.dot(e, w1, preferred_element_type=jnp.float32) + b1
        z1 = _leaky(y1 * a1 + c1)
        y2 = jnp.dot(z1, w2, preferred_element_type=jnp.float32)
        acc = jnp.maximum(acc, _leaky(y2 * a2 + c2))
    out_ref[...] = acc


def _edge_conv_sc(x, idx, lay1, lay2):
    n, c = x.shape
    w1, b1, g1, be1 = lay1
    w2, b2, g2, be2 = lay2
    h = w1.shape[1]
    o = w2.shape[1]
    cp = ((c + 127) // 128) * 128
    xp = jnp.pad(x, ((0, _NP - n), (0, cp - c)))
    w1cat = jnp.zeros((2 * cp, h), jnp.float32)
    w1cat = w1cat.at[:c].set(w1[:c]).at[cp:cp + c].set(w1[c:])
    b1r = b1[None, :]
    idxp = jnp.pad(idx, ((0, _NP - n), (0, 0)))
    idx3 = idxp.T.reshape(_K * (_NP // 256), 2, 128)

    g3 = _sc_gather(xp, idx3, cp)

    grid = _NP // _R
    full3 = pl.BlockSpec((_K, _R, cp), lambda g: (0, g, 0))
    rowb = pl.BlockSpec((_R, cp), lambda g: (g, 0))
    vec = lambda d: pl.BlockSpec((1, d), lambda g: (0, 0))
    mat = lambda a, b: pl.BlockSpec((a, b), lambda g: (0, 0))

    s1, q1 = pl.pallas_call(
        functools.partial(_e_stats1_body, cp, h),
        grid=(grid,),
        in_specs=[full3, rowb, mat(2 * cp, h), vec(h)],
        out_specs=pl.BlockSpec((1, 2, h), lambda g: (g, 0, 0)),
        out_shape=jax.ShapeDtypeStruct((grid, 2, h), jnp.float32),
    )(g3, xp, w1cat, b1r).sum(axis=0)

    cnt = jnp.float32(n * _K)
    m1 = s1 / cnt
    v1 = q1 / cnt - m1 * m1
    sc1 = g1 / jnp.sqrt(v1 + 1e-5)
    a1 = sc1[None, :]
    c1f = (be1 - m1 * sc1)[None, :]

    s2, q2 = pl.pallas_call(
        functools.partial(_e_stats2_body, cp, h, o),
        grid=(grid,),
        in_specs=[full3, rowb, mat(2 * cp, h), vec(h), vec(h), vec(h),
                  mat(h, o)],
        out_specs=pl.BlockSpec((1, 2, o), lambda g: (g, 0, 0)),
        out_shape=jax.ShapeDtypeStruct((grid, 2, o), jnp.float32),
    )(g3, xp, w1cat, b1r, a1, c1f, w2).sum(axis=0)

    m2nb = s2 / cnt
    v2 = q2 / cnt - m2nb * m2nb
    sc2 = g2 / jnp.sqrt(v2 + 1e-5)
    a2 = sc2[None, :]
    c2f = ((b2 - (m2nb + b2)) * sc2 + be2)[None, :]

    out = pl.pallas_call(
        functools.partial(_e_final_body, cp, h, o),
        grid=(grid,),
        in_specs=[full3, rowb, mat(2 * cp, h), vec(h), vec(h), vec(h),
                  mat(h, o), vec(o), vec(o)],
        out_specs=pl.BlockSpec((_R, o), lambda g: (g, 0)),
        out_shape=jax.ShapeDtypeStruct((_NP, o), jnp.float32),
    )(g3, xp, w1cat, b1r, a1, c1f, w2, a2, c2f)
    return out[:n]


def _leaky(x):
    return jnp.where(x >= 0, x, 0.2 * x)


def _bn(x, g, b):
    m = jnp.mean(x, axis=0)
    v = jnp.var(x, axis=0)
    return (x - m) / jnp.sqrt(v + 1e-5) * g + b


def _mlp(x, layers, use_bn):
    for lay in layers:
        if use_bn:
            w, b, g, be = lay
            x = _leaky(_bn(x @ w + b, g, be))
        else:
            w, b = lay
            x = _leaky(x @ w + b)
    return x


def _gather_neighbors(x, idx):
    n, c = x.shape
    cp = ((c + 127) // 128) * 128
    xp = jnp.pad(x, ((0, _NP - n), (0, cp - c)))
    idxp = jnp.pad(idx, ((0, _NP - n), (0, 0)))
    idx3 = idxp.T.reshape(_K * (_NP // 256), 2, 128)
    g3 = _sc_gather(xp, idx3, cp)
    return g3[:, :n, :c].transpose(1, 0, 2)


def _edge_conv(x, batch, layers):
    idx = _knn_idx_pallas(x, batch)
    x_j = _gather_neighbors(x, idx)
    x_i = jnp.broadcast_to(x[:, None, :], x_j.shape)
    hh = jnp.concatenate([x_i, x_j - x_i], axis=-1)
    n, k, c = hh.shape
    hh = _mlp(hh.reshape(n * k, c), layers, True)
    return jnp.max(hh.reshape(n, k, -1), axis=1)


def kernel(positions, features, batch_indices, params):
    n_layers, n_batch = 2, 8
    x = _edge_conv(positions, batch_indices, params["t1"])
    x = _mlp(x, params["t2"], True)
    x = jax.ops.segment_max(x, batch_indices, num_segments=n_batch)
    x = _mlp(x, params["t3"], False)
    w4, b4 = params["t4"]
    x = x @ w4 + b4
    x = x[batch_indices].reshape(-1, 3, 3)
    x0 = jnp.einsum('ni,nij->nj', positions, x)
    x = jnp.concatenate([x0, features], axis=-1)
    for i in range(n_layers):
        x_i = _edge_conv(x, batch_indices, params["convs"][i])
        (w1, b1), (w2, b2) = params["lins"][i]
        x_i = jnp.maximum(x_i @ w1 + b1, 0.0) @ w2 + b2
        wt, bt = params["ltrans"][i]
        x = (x @ wt + bt) + x_i
    return x


# speedup vs baseline: 1.4102x; 1.1708x over previous
import functools
import math

import jax
import jax.numpy as jnp
from jax.experimental import pallas as pl
from jax.experimental.pallas import tpu as pltpu
from jax.experimental.pallas import tpu_sc as plsc

_N = 10000
_NP = 10240
_R = 256
_CB = 1024
_NCH = _NP // _CB
_K = 20


def _knn_body(c0_ref, c1_ref, xr_ref, br_ref, xc_ref, sqc_ref, bc_ref,
              idx_ref, d_ref, m_ref):
    g = pl.program_id(0)
    c0 = c0_ref[g]
    c1 = c1_ref[g]
    xr = xr_ref[...]
    sqr = jnp.sum(xr * xr, axis=1, keepdims=True)
    br = br_ref[...]
    m_ref[...] = jnp.full((_NCH, _R, 1), jnp.inf, jnp.float32)

    def fill(ci, carry):
        xc = xc_ref[ci]
        dd = sqr + sqc_ref[ci] - 2.0 * jnp.dot(
            xr, xc, preferred_element_type=jnp.float32)
        dd = jnp.where(br == bc_ref[ci], dd, jnp.inf)
        d_ref[ci] = dd
        m_ref[ci] = jnp.min(dd, axis=1, keepdims=True)
        return carry

    jax.lax.fori_loop(c0, c1, fill, 0)

    iotaf = jax.lax.broadcasted_iota(
        jnp.int32, (1, _CB), 1).astype(jnp.float32)
    bigf = jnp.float32(2.0 * _NP)
    outs = []
    for _ in range(_K):
        cm = jnp.min(m_ref[...], axis=0)

        def extract(ci, bi):
            d = d_ref[ci]
            match = d == cm
            iif = iotaf + ci.astype(jnp.float32) * _CB
            im_c = jnp.min(jnp.where(match, iif, bigf), axis=1, keepdims=True)
            is_first = bi >= bigf
            dnew = jnp.where((iif == im_c) & is_first, jnp.inf, d)
            d_ref[ci] = dnew
            m_ref[ci] = jnp.min(dnew, axis=1, keepdims=True)
            return jnp.minimum(bi, im_c)

        bi = jax.lax.fori_loop(c0, c1, extract, jnp.full((_R, 1), bigf))
        outs.append(bi)
    idx = jnp.concatenate(outs, axis=1).astype(jnp.int32)
    idx_ref[...] = jnp.minimum(idx, _N - 1)


def _knn_idx_pallas(x, batch):
    n, c = x.shape
    cp = max(8, ((c + 7) // 8) * 8)
    xp = jnp.zeros((_NP, cp), jnp.float32).at[:n, :c].set(x)
    batp = jnp.concatenate(
        [batch.astype(jnp.int32), jnp.full((_NP - n,), 8, jnp.int32)])
    batf = batp.astype(jnp.float32)

    starts9 = jnp.searchsorted(batp, jnp.arange(9, dtype=jnp.int32), side="left")
    ends9 = jnp.searchsorted(batp, jnp.arange(9, dtype=jnp.int32), side="right")
    b_lo = batp[0::_R]
    b_hi = batp[_R - 1::_R]
    cs = starts9[b_lo]
    ce = ends9[b_hi]
    c0 = (cs // _CB).astype(jnp.int32)
    c1 = ((ce + _CB - 1) // _CB).astype(jnp.int32)

    x3t = xp.reshape(_NCH, _CB, cp).transpose(0, 2, 1)
    sqc3 = jnp.sum(xp * xp, axis=1).reshape(_NCH, _CB)[:, None, :]
    bc3 = batf.reshape(_NCH, _CB)[:, None, :]
    br2 = batf[:, None]

    grid = _NP // _R
    idx = pl.pallas_call(
        _knn_body,
        grid=(grid,),
        in_specs=[
            pl.BlockSpec(memory_space=pltpu.SMEM),
            pl.BlockSpec(memory_space=pltpu.SMEM),
            pl.BlockSpec((_R, cp), lambda g: (g, 0)),
            pl.BlockSpec((_R, 1), lambda g: (g, 0)),
            pl.BlockSpec((_NCH, cp, _CB), lambda g: (0, 0, 0)),
            pl.BlockSpec((_NCH, 1, _CB), lambda g: (0, 0, 0)),
            pl.BlockSpec((_NCH, 1, _CB), lambda g: (0, 0, 0)),
        ],
        out_specs=pl.BlockSpec((_R, _K), lambda g: (g, 0)),
        out_shape=jax.ShapeDtypeStruct((_NP, _K), jnp.int32),
        scratch_shapes=[pltpu.VMEM((_NCH, _R, _CB), jnp.float32),
                        pltpu.VMEM((_NCH, _R, 1), jnp.float32)],
    )(c0, c1, xp, br2, x3t, sqc3, bc3)
    return idx[:n]


def _sc_gather(table, idx3, h):
    n_st = _NP // 256
    n_super = _K * n_st
    per_w = n_super // 32
    mesh = plsc.VectorSubcoreMesh(core_axis_name="c", subcore_axis_name="s")

    @functools.partial(
        pl.kernel, mesh=mesh,
        out_type=jax.ShapeDtypeStruct((_K, _NP, h), jnp.float32),
        scratch_types=[
            pltpu.VMEM((2, 128), jnp.int32),
            pltpu.VMEM((256, h), jnp.float32),
            pltpu.SemaphoreType.DMA,
        ],
    )
    def gk(table_h, idx_h, out_h, idx_v, rows_v, sem):
        wid = jax.lax.axis_index("s") * 2 + jax.lax.axis_index("c")

        def body(j, carry):
            st = wid * per_w + j
            k = st // n_st
            pos = (st % n_st) * 256
            pltpu.sync_copy(idx_h.at[st], idx_v)
            cps = [
                pltpu.async_copy(
                    table_h.at[idx_v.at[i]],
                    rows_v.at[pl.ds(i * 128, 128)], sem)
                for i in range(2)
            ]
            for cp in cps:
                cp.wait()
            pltpu.sync_copy(rows_v, out_h.at[k, pl.ds(pos, 256)])
            return carry

        jax.lax.fori_loop(0, per_w, body, 0)

    return gk(table, idx3)


def _e_stats1_body(cp, h, g_ref, x_ref, w1_ref, b1_ref, o_ref):
    g = pl.program_id(0)
    xi = x_ref[...]
    w1 = w1_ref[...]
    b1 = b1_ref[...]
    rows = jax.lax.broadcasted_iota(jnp.int32, (_R, 1), 0) + g * _R
    vm = rows < _N
    s = jnp.zeros((1, h), jnp.float32)
    q = jnp.zeros((1, h), jnp.float32)
    for k in range(_K):
        e = jnp.concatenate([xi, g_ref[k] - xi], axis=1)
        y = jnp.dot(e, w1, preferred_element_type=jnp.float32) + b1
        y = jnp.where(vm, y, 0.0)
        s = s + jnp.sum(y, axis=0, keepdims=True)
        q = q + jnp.sum(y * y, axis=0, keepdims=True)
    o_ref[0, 0:1, :] = s
    o_ref[0, 1:2, :] = q


def _e_stats2_body(cp, h, o, g_ref, x_ref, w1_ref, b1_ref, a1_ref, c1_ref,
                   w2_ref, o_ref):
    g = pl.program_id(0)
    xi = x_ref[...]
    w1 = w1_ref[...]
    b1 = b1_ref[...]
    a1 = a1_ref[...]
    c1 = c1_ref[...]
    w2 = w2_ref[...]
    rows = jax.lax.broadcasted_iota(jnp.int32, (_R, 1), 0) + g * _R
    vm = rows < _N
    s = jnp.zeros((1, o), jnp.float32)
    q = jnp.zeros((1, o), jnp.float32)
    for k in range(_K):
        e = jnp.concatenate([xi, g_ref[k] - xi], axis=1)
        y1 = jnp.dot(e, w1, preferred_element_type=jnp.float32) + b1
        z1 = _leaky(y1 * a1 + c1)
        y2 = jnp.where(vm, jnp.dot(z1, w2, preferred_element_type=jnp.float32),
                       0.0)
        s = s + jnp.sum(y2, axis=0, keepdims=True)
        q = q + jnp.sum(y2 * y2, axis=0, keepdims=True)
    o_ref[0, 0:1, :] = s
    o_ref[0, 1:2, :] = q


def _e_final_body(cp, h, o, g_ref, x_ref, w1_ref, b1_ref, a1_ref, c1_ref,
                  w2_ref, a2_ref, c2_ref, out_ref):
    xi = x_ref[...]
    w1 = w1_ref[...]
    b1 = b1_ref[...]
    a1 = a1_ref[...]
    c1 = c1_ref[...]
    w2 = w2_ref[...]
    a2 = a2_ref[...]
    c2 = c2_ref[...]
    acc = jnp.full((_R, o), -jnp.inf, jnp.float32)
    for k in range(_K):
        e = jnp.concatenate([xi, g_ref[k] - xi], axis=1)
        y1 = jnp.dot(e, w1, preferred_element_type=jnp.float32) + b1
        z1 = _leaky(y1 * a1 + c1)
        y2 = jnp.dot(z1, w2, preferred_element_type=jnp.float32)
        acc = jnp.maximum(acc, _leaky(y2 * a2 + c2))
    out_ref[...] = acc


def _edge_conv_sc(x, idx, lay1, lay2):
    n, c = x.shape
    w1, b1, g1, be1 = lay1
    w2, b2, g2, be2 = lay2
    h = w1.shape[1]
    o = w2.shape[1]
    cp = ((c + 127) // 128) * 128
    xp = jnp.pad(x, ((0, _NP - n), (0, cp - c)))
    w1cat = jnp.zeros((2 * cp, h), jnp.float32)
    w1cat = w1cat.at[:c].set(w1[:c]).at[cp:cp + c].set(w1[c:])
    b1r = b1[None, :]
    idxp = jnp.pad(idx, ((0, _NP - n), (0, 0)))
    idx3 = idxp.T.reshape(_K * (_NP // 256), 2, 128)

    g3 = _sc_gather(xp, idx3, cp)

    grid = _NP // _R
    full3 = pl.BlockSpec((_K, _R, cp), lambda g: (0, g, 0))
    rowb = pl.BlockSpec((_R, cp), lambda g: (g, 0))
    vec = lambda d: pl.BlockSpec((1, d), lambda g: (0, 0))
    mat = lambda a, b: pl.BlockSpec((a, b), lambda g: (0, 0))

    s1, q1 = pl.pallas_call(
        functools.partial(_e_stats1_body, cp, h),
        grid=(grid,),
        in_specs=[full3, rowb, mat(2 * cp, h), vec(h)],
        out_specs=pl.BlockSpec((1, 2, h), lambda g: (g, 0, 0)),
        out_shape=jax.ShapeDtypeStruct((grid, 2, h), jnp.float32),
    )(g3, xp, w1cat, b1r).sum(axis=0)

    cnt = jnp.float32(n * _K)
    m1 = s1 / cnt
    v1 = q1 / cnt - m1 * m1
    sc1 = g1 / jnp.sqrt(v1 + 1e-5)
    a1 = sc1[None, :]
    c1f = (be1 - m1 * sc1)[None, :]

    s2, q2 = pl.pallas_call(
        functools.partial(_e_stats2_body, cp, h, o),
        grid=(grid,),
        in_specs=[full3, rowb, mat(2 * cp, h), vec(h), vec(h), vec(h),
                  mat(h, o)],
        out_specs=pl.BlockSpec((1, 2, o), lambda g: (g, 0, 0)),
        out_shape=jax.ShapeDtypeStruct((grid, 2, o), jnp.float32),
    )(g3, xp, w1cat, b1r, a1, c1f, w2).sum(axis=0)

    m2nb = s2 / cnt
    v2 = q2 / cnt - m2nb * m2nb
    sc2 = g2 / jnp.sqrt(v2 + 1e-5)
    a2 = sc2[None, :]
    c2f = ((b2 - (m2nb + b2)) * sc2 + be2)[None, :]

    out = pl.pallas_call(
        functools.partial(_e_final_body, cp, h, o),
        grid=(grid,),
        in_specs=[full3, rowb, mat(2 * cp, h), vec(h), vec(h), vec(h),
                  mat(h, o), vec(o), vec(o)],
        out_specs=pl.BlockSpec((_R, o), lambda g: (g, 0)),
        out_shape=jax.ShapeDtypeStruct((_NP, o), jnp.float32),
    )(g3, xp, w1cat, b1r, a1, c1f, w2, a2, c2f)
    return out[:n]


def _leaky(x):
    return jnp.where(x >= 0, x, 0.2 * x)


def _bn(x, g, b):
    m = jnp.mean(x, axis=0)
    v = jnp.var(x, axis=0)
    return (x - m) / jnp.sqrt(v + 1e-5) * g + b


def _mlp(x, layers, use_bn):
    for lay in layers:
        if use_bn:
            w, b, g, be = lay
            x = _leaky(_bn(x @ w + b, g, be))
        else:
            w, b = lay
            x = _leaky(x @ w + b)
    return x


def _gather_neighbors(x, idx):
    n, c = x.shape
    cp = ((c + 127) // 128) * 128
    xp = jnp.pad(x, ((0, _NP - n), (0, cp - c)))
    idxp = jnp.pad(idx, ((0, _NP - n), (0, 0)))
    idx3 = idxp.T.reshape(_K * (_NP // 256), 2, 128)
    g3 = _sc_gather(xp, idx3, cp)
    return g3[:, :n, :c].transpose(1, 0, 2)


def _edge_conv(x, batch, layers):
    idx = _knn_idx_pallas(x, batch)
    x_j = _gather_neighbors(x, idx)
    x_i = jnp.broadcast_to(x[:, None, :], x_j.shape)
    hh = jnp.concatenate([x_i, x_j - x_i], axis=-1)
    n, k, c = hh.shape
    hh = _mlp(hh.reshape(n * k, c), layers, True)
    return jnp.max(hh.reshape(n, k, -1), axis=1)


def kernel(positions, features, batch_indices, params):
    n_layers, n_batch = 2, 8
    x = _edge_conv(positions, batch_indices, params["t1"])
    x = _mlp(x, params["t2"], True)
    x = jax.ops.segment_max(x, batch_indices, num_segments=n_batch)
    x = _mlp(x, params["t3"], False)
    w4, b4 = params["t4"]
    x = x @ w4 + b4
    x = x[batch_indices].reshape(-1, 3, 3)
    x0 = jnp.einsum('ni,nij->nj', positions, x)
    x = jnp.concatenate([x0, features], axis=-1)
    for i in range(n_layers):
        x_i = _edge_conv(x, batch_indices, params["convs"][i])
        (w1, b1), (w2, b2) = params["lins"][i]
        x_i = jnp.maximum(x_i @ w1 + b1, 0.0) @ w2 + b2
        wt, bt = params["ltrans"][i]
        x = (x @ wt + bt) + x_i
    return x


# speedup vs baseline: 1.4642x; 1.0383x over previous
import functools
import math

import jax
import jax.numpy as jnp
from jax.experimental import pallas as pl
from jax.experimental.pallas import tpu as pltpu
from jax.experimental.pallas import tpu_sc as plsc

_N = 10000
_NP = 10240
_R = 512
_CB = 1024
_NCH = _NP // _CB
_K = 20


def _knn_body(c0_ref, c1_ref, xr_ref, br_ref, xc_ref, sqc_ref, bc_ref,
              idx_ref, d_ref, m_ref):
    g = pl.program_id(0)
    c0 = c0_ref[g]
    c1 = c1_ref[g]
    xr = xr_ref[...]
    sqr = jnp.sum(xr * xr, axis=1, keepdims=True)
    br = br_ref[...]
    m_ref[...] = jnp.full((_NCH, _R, 1), jnp.inf, jnp.float32)

    def fill(ci, carry):
        xc = xc_ref[ci]
        dd = sqr + sqc_ref[ci] - 2.0 * jnp.dot(
            xr, xc, preferred_element_type=jnp.float32)
        dd = jnp.where(br == bc_ref[ci], dd, jnp.inf)
        d_ref[ci] = dd
        m_ref[ci] = jnp.min(dd, axis=1, keepdims=True)
        return carry

    jax.lax.fori_loop(c0, c1, fill, 0)

    iotaf = jax.lax.broadcasted_iota(
        jnp.int32, (1, _CB), 1).astype(jnp.float32)
    bigf = jnp.float32(2.0 * _NP)
    outs = []
    for _ in range(_K):
        cm = jnp.min(m_ref[...], axis=0)

        def extract(ci, bi):
            d = d_ref[ci]
            match = d == cm
            iif = iotaf + ci.astype(jnp.float32) * _CB
            im_c = jnp.min(jnp.where(match, iif, bigf), axis=1, keepdims=True)
            is_first = bi >= bigf
            dnew = jnp.where((iif == im_c) & is_first, jnp.inf, d)
            d_ref[ci] = dnew
            m_ref[ci] = jnp.min(dnew, axis=1, keepdims=True)
            return jnp.minimum(bi, im_c)

        bi = jax.lax.fori_loop(c0, c1, extract, jnp.full((_R, 1), bigf))
        outs.append(bi)
    idx = jnp.concatenate(outs, axis=1).astype(jnp.int32)
    idx_ref[...] = jnp.minimum(idx, _N - 1)


def _knn_idx_pallas(x, batch):
    n, c = x.shape
    cp = max(8, ((c + 7) // 8) * 8)
    xp = jnp.zeros((_NP, cp), jnp.float32).at[:n, :c].set(x)
    batp = jnp.concatenate(
        [batch.astype(jnp.int32), jnp.full((_NP - n,), 8, jnp.int32)])
    batf = batp.astype(jnp.float32)

    starts9 = jnp.searchsorted(batp, jnp.arange(9, dtype=jnp.int32), side="left")
    ends9 = jnp.searchsorted(batp, jnp.arange(9, dtype=jnp.int32), side="right")
    b_lo = batp[0::_R]
    b_hi = batp[_R - 1::_R]
    cs = starts9[b_lo]
    ce = ends9[b_hi]
    c0 = (cs // _CB).astype(jnp.int32)
    c1 = ((ce + _CB - 1) // _CB).astype(jnp.int32)

    x3t = xp.reshape(_NCH, _CB, cp).transpose(0, 2, 1)
    sqc3 = jnp.sum(xp * xp, axis=1).reshape(_NCH, _CB)[:, None, :]
    bc3 = batf.reshape(_NCH, _CB)[:, None, :]
    br2 = batf[:, None]

    grid = _NP // _R
    idx = pl.pallas_call(
        _knn_body,
        grid=(grid,),
        in_specs=[
            pl.BlockSpec(memory_space=pltpu.SMEM),
            pl.BlockSpec(memory_space=pltpu.SMEM),
            pl.BlockSpec((_R, cp), lambda g: (g, 0)),
            pl.BlockSpec((_R, 1), lambda g: (g, 0)),
            pl.BlockSpec((_NCH, cp, _CB), lambda g: (0, 0, 0)),
            pl.BlockSpec((_NCH, 1, _CB), lambda g: (0, 0, 0)),
            pl.BlockSpec((_NCH, 1, _CB), lambda g: (0, 0, 0)),
        ],
        out_specs=pl.BlockSpec((_R, _K), lambda g: (g, 0)),
        out_shape=jax.ShapeDtypeStruct((_NP, _K), jnp.int32),
        scratch_shapes=[pltpu.VMEM((_NCH, _R, _CB), jnp.float32),
                        pltpu.VMEM((_NCH, _R, 1), jnp.float32)],
    )(c0, c1, xp, br2, x3t, sqc3, bc3)
    return idx[:n]


def _sc_gather(table, idx3, h):
    n_st = _NP // 256
    n_super = _K * n_st
    per_w = n_super // 32
    mesh = plsc.VectorSubcoreMesh(core_axis_name="c", subcore_axis_name="s")

    @functools.partial(
        pl.kernel, mesh=mesh,
        out_type=jax.ShapeDtypeStruct((_K, _NP, h), jnp.float32),
        scratch_types=[
            pltpu.VMEM((2, 128), jnp.int32),
            pltpu.VMEM((256, h), jnp.float32),
            pltpu.SemaphoreType.DMA,
        ],
    )
    def gk(table_h, idx_h, out_h, idx_v, rows_v, sem):
        wid = jax.lax.axis_index("s") * 2 + jax.lax.axis_index("c")

        def body(j, carry):
            st = wid * per_w + j
            k = st // n_st
            pos = (st % n_st) * 256
            pltpu.sync_copy(idx_h.at[st], idx_v)
            cps = [
                pltpu.async_copy(
                    table_h.at[idx_v.at[i]],
                    rows_v.at[pl.ds(i * 128, 128)], sem)
                for i in range(2)
            ]
            for cp in cps:
                cp.wait()
            pltpu.sync_copy(rows_v, out_h.at[k, pl.ds(pos, 256)])
            return carry

        jax.lax.fori_loop(0, per_w, body, 0)

    return gk(table, idx3)


def _e_stats1_body(cp, h, g_ref, x_ref, w1_ref, b1_ref, o_ref):
    g = pl.program_id(0)
    xi = x_ref[...]
    w1 = w1_ref[...]
    b1 = b1_ref[...]
    rows = jax.lax.broadcasted_iota(jnp.int32, (_R, 1), 0) + g * _R
    vm = rows < _N
    s = jnp.zeros((1, h), jnp.float32)
    q = jnp.zeros((1, h), jnp.float32)
    for k in range(_K):
        e = jnp.concatenate([xi, g_ref[k] - xi], axis=1)
        y = jnp.dot(e, w1, preferred_element_type=jnp.float32) + b1
        y = jnp.where(vm, y, 0.0)
        s = s + jnp.sum(y, axis=0, keepdims=True)
        q = q + jnp.sum(y * y, axis=0, keepdims=True)
    o_ref[0, 0:1, :] = s
    o_ref[0, 1:2, :] = q


def _e_stats2_body(cp, h, o, g_ref, x_ref, w1_ref, b1_ref, a1_ref, c1_ref,
                   w2_ref, o_ref):
    g = pl.program_id(0)
    xi = x_ref[...]
    w1 = w1_ref[...]
    b1 = b1_ref[...]
    a1 = a1_ref[...]
    c1 = c1_ref[...]
    w2 = w2_ref[...]
    rows = jax.lax.broadcasted_iota(jnp.int32, (_R, 1), 0) + g * _R
    vm = rows < _N
    s = jnp.zeros((1, o), jnp.float32)
    q = jnp.zeros((1, o), jnp.float32)
    for k in range(_K):
        e = jnp.concatenate([xi, g_ref[k] - xi], axis=1)
        y1 = jnp.dot(e, w1, preferred_element_type=jnp.float32) + b1
        z1 = _leaky(y1 * a1 + c1)
        y2 = jnp.where(vm, jnp.dot(z1, w2, preferred_element_type=jnp.float32),
                       0.0)
        s = s + jnp.sum(y2, axis=0, keepdims=True)
        q = q + jnp.sum(y2 * y2, axis=0, keepdims=True)
    o_ref[0, 0:1, :] = s
    o_ref[0, 1:2, :] = q


def _e_final_body(cp, h, o, g_ref, x_ref, w1_ref, b1_ref, a1_ref, c1_ref,
                  w2_ref, a2_ref, c2_ref, out_ref):
    xi = x_ref[...]
    w1 = w1_ref[...]
    b1 = b1_ref[...]
    a1 = a1_ref[...]
    c1 = c1_ref[...]
    w2 = w2_ref[...]
    a2 = a2_ref[...]
    c2 = c2_ref[...]
    acc = jnp.full((_R, o), -jnp.inf, jnp.float32)
    for k in range(_K):
        e = jnp.concatenate([xi, g_ref[k] - xi], axis=1)
        y1 = jnp.dot(e, w1, preferred_element_type=jnp.float32) + b1
        z1 = _leaky(y1 * a1 + c1)
        y2 = jnp.dot(z1, w2, preferred_element_type=jnp.float32)
        acc = jnp.maximum(acc, _leaky(y2 * a2 + c2))
    out_ref[...] = acc


def _edge_conv_sc(x, idx, lay1, lay2):
    n, c = x.shape
    w1, b1, g1, be1 = lay1
    w2, b2, g2, be2 = lay2
    h = w1.shape[1]
    o = w2.shape[1]
    cp = ((c + 127) // 128) * 128
    xp = jnp.pad(x, ((0, _NP - n), (0, cp - c)))
    w1cat = jnp.zeros((2 * cp, h), jnp.float32)
    w1cat = w1cat.at[:c].set(w1[:c]).at[cp:cp + c].set(w1[c:])
    b1r = b1[None, :]
    idxp = jnp.pad(idx, ((0, _NP - n), (0, 0)))
    idx3 = idxp.T.reshape(_K * (_NP // 256), 2, 128)

    g3 = _sc_gather(xp, idx3, cp)

    grid = _NP // _R
    full3 = pl.BlockSpec((_K, _R, cp), lambda g: (0, g, 0))
    rowb = pl.BlockSpec((_R, cp), lambda g: (g, 0))
    vec = lambda d: pl.BlockSpec((1, d), lambda g: (0, 0))
    mat = lambda a, b: pl.BlockSpec((a, b), lambda g: (0, 0))

    s1, q1 = pl.pallas_call(
        functools.partial(_e_stats1_body, cp, h),
        grid=(grid,),
        in_specs=[full3, rowb, mat(2 * cp, h), vec(h)],
        out_specs=pl.BlockSpec((1, 2, h), lambda g: (g, 0, 0)),
        out_shape=jax.ShapeDtypeStruct((grid, 2, h), jnp.float32),
    )(g3, xp, w1cat, b1r).sum(axis=0)

    cnt = jnp.float32(n * _K)
    m1 = s1 / cnt
    v1 = q1 / cnt - m1 * m1
    sc1 = g1 / jnp.sqrt(v1 + 1e-5)
    a1 = sc1[None, :]
    c1f = (be1 - m1 * sc1)[None, :]

    s2, q2 = pl.pallas_call(
        functools.partial(_e_stats2_body, cp, h, o),
        grid=(grid,),
        in_specs=[full3, rowb, mat(2 * cp, h), vec(h), vec(h), vec(h),
                  mat(h, o)],
        out_specs=pl.BlockSpec((1, 2, o), lambda g: (g, 0, 0)),
        out_shape=jax.ShapeDtypeStruct((grid, 2, o), jnp.float32),
    )(g3, xp, w1cat, b1r, a1, c1f, w2).sum(axis=0)

    m2nb = s2 / cnt
    v2 = q2 / cnt - m2nb * m2nb
    sc2 = g2 / jnp.sqrt(v2 + 1e-5)
    a2 = sc2[None, :]
    c2f = ((b2 - (m2nb + b2)) * sc2 + be2)[None, :]

    out = pl.pallas_call(
        functools.partial(_e_final_body, cp, h, o),
        grid=(grid,),
        in_specs=[full3, rowb, mat(2 * cp, h), vec(h), vec(h), vec(h),
                  mat(h, o), vec(o), vec(o)],
        out_specs=pl.BlockSpec((_R, o), lambda g: (g, 0)),
        out_shape=jax.ShapeDtypeStruct((_NP, o), jnp.float32),
    )(g3, xp, w1cat, b1r, a1, c1f, w2, a2, c2f)
    return out[:n]


def _leaky(x):
    return jnp.where(x >= 0, x, 0.2 * x)


def _bn(x, g, b):
    m = jnp.mean(x, axis=0)
    v = jnp.var(x, axis=0)
    return (x - m) / jnp.sqrt(v + 1e-5) * g + b


def _mlp(x, layers, use_bn):
    for lay in layers:
        if use_bn:
            w, b, g, be = lay
            x = _leaky(_bn(x @ w + b, g, be))
        else:
            w, b = lay
            x = _leaky(x @ w + b)
    return x


def _gather_neighbors(x, idx):
    n, c = x.shape
    cp = ((c + 127) // 128) * 128
    xp = jnp.pad(x, ((0, _NP - n), (0, cp - c)))
    idxp = jnp.pad(idx, ((0, _NP - n), (0, 0)))
    idx3 = idxp.T.reshape(_K * (_NP // 256), 2, 128)
    g3 = _sc_gather(xp, idx3, cp)
    return g3[:, :n, :c].transpose(1, 0, 2)


def _edge_conv(x, batch, layers):
    idx = _knn_idx_pallas(x, batch)
    x_j = _gather_neighbors(x, idx)
    x_i = jnp.broadcast_to(x[:, None, :], x_j.shape)
    hh = jnp.concatenate([x_i, x_j - x_i], axis=-1)
    n, k, c = hh.shape
    hh = _mlp(hh.reshape(n * k, c), layers, True)
    return jnp.max(hh.reshape(n, k, -1), axis=1)


def kernel(positions, features, batch_indices, params):
    n_layers, n_batch = 2, 8
    x = _edge_conv(positions, batch_indices, params["t1"])
    x = _mlp(x, params["t2"], True)
    x = jax.ops.segment_max(x, batch_indices, num_segments=n_batch)
    x = _mlp(x, params["t3"], False)
    w4, b4 = params["t4"]
    x = x @ w4 + b4
    x = x[batch_indices].reshape(-1, 3, 3)
    x0 = jnp.einsum('ni,nij->nj', positions, x)
    x = jnp.concatenate([x0, features], axis=-1)
    for i in range(n_layers):
        x_i = _edge_conv(x, batch_indices, params["convs"][i])
        (w1, b1), (w2, b2) = params["lins"][i]
        x_i = jnp.maximum(x_i @ w1 + b1, 0.0) @ w2 + b2
        wt, bt = params["ltrans"][i]
        x = (x @ wt + bt) + x_i
    return x
